# Initial kernel scaffold; baseline (speedup 1.0000x reference)
import jax
import jax.numpy as jnp
from jax.experimental import pallas as pl


def kernel(x_cat, tables, bias):
    raise NotImplementedError("write your pallas kernel here")


# speedup vs baseline: 6.0995x; 6.0995x over previous
import functools

import jax
import jax.numpy as jnp
from jax import lax
from jax.experimental import pallas as pl
from jax.experimental.pallas import tpu as pltpu
from jax.experimental.pallas import tpu_sc as plsc

F = 26
CARD = 1000
D = 128
B = 4096
L = 16
NC, NS = 2, 16
NW = NC * NS
ROWS = B * F
RPW = ROWS // NW
CHUNK = 128
NCH = RPW // CHUNK
VPR = D // L

_mesh = plsc.VectorSubcoreMesh(core_axis_name="c", subcore_axis_name="s")


@functools.partial(
    pl.kernel,
    out_type=jax.ShapeDtypeStruct((ROWS, D), jnp.float32),
    mesh=_mesh,
    scratch_types=[
        pltpu.VMEM((CHUNK,), jnp.int32),
        pltpu.VMEM((CHUNK,), jnp.int32),
        pltpu.VMEM((CHUNK, D), jnp.float32),
        pltpu.VMEM((F, D), jnp.float32),
        pltpu.SemaphoreType.DMA,
    ],
)
def _tokenize(idx_hbm, tab_hbm, bias_hbm, out_hbm, raw_v, gid_v, buf_v, bias_v, sem):
    wid = lax.axis_index("s") * NC + lax.axis_index("c")
    wbase = wid * RPW
    pltpu.sync_copy(bias_hbm, bias_v)
    lane = lax.iota(jnp.int32, L)

    for k in range(NCH):
        base = wbase + k * CHUNK
        pltpu.sync_copy(idx_hbm.at[pl.ds(base, CHUNK)], raw_v)
        for j in range(CHUNK // L):
            pos = base + j * L + lane
            v = jnp.maximum(raw_v[pl.ds(j * L, L)], 0)
            gid_v[pl.ds(j * L, L)] = v + lax.rem(pos, F) * CARD
        pltpu.async_copy(tab_hbm.at[gid_v], buf_v, sem).wait()

        @pl.loop(0, CHUNK)
        def _bias_add(r):
            fr = lax.rem(base + r, F)
            for j in range(VPR):
                sl = pl.ds(j * L, L)
                buf_v[r, sl] = buf_v[r, sl] + bias_v[fr, sl]

        pltpu.sync_copy(buf_v, out_hbm.at[pl.ds(base, CHUNK), :])


def kernel(x_cat, tables, bias):
    idx = x_cat.astype(jnp.int32).reshape(ROWS)
    tab = tables.reshape(F * CARD, D)
    out = _tokenize(idx, tab, bias)
    return out.reshape(B, F, D)


# speedup vs baseline: 7.5208x; 1.2330x over previous
import functools

import jax
import jax.numpy as jnp
from jax import lax
from jax.experimental import pallas as pl
from jax.experimental.pallas import tpu as pltpu
from jax.experimental.pallas import tpu_sc as plsc

F = 26
CARD = 1000
D = 128
B = 4096
L = 16
NC, NS = 2, 16
NW = NC * NS
ROWS = B * F
RPW = ROWS // NW
CHUNK = 128
NCH = RPW // CHUNK
VPR = D // L
NB = 4

_mesh = plsc.VectorSubcoreMesh(core_axis_name="c", subcore_axis_name="s")


@functools.partial(
    pl.kernel,
    out_type=jax.ShapeDtypeStruct((ROWS, D), jnp.float32),
    mesh=_mesh,
    scratch_types=[
        pltpu.VMEM((RPW,), jnp.int32),
        pltpu.VMEM((NCH, CHUNK), jnp.int32),
        pltpu.VMEM((F, D), jnp.float32),
    ]
    + [pltpu.VMEM((CHUNK, D), jnp.float32) for _ in range(NB)]
    + [pltpu.SemaphoreType.DMA for _ in range(2 * NB)],
)
def _tokenize(idx_hbm, tab_hbm, bias_hbm, out_hbm, raw_v, gid_v, bias_v, *bufs_sems):
    bufq = bufs_sems[:NB]
    sem_g = bufs_sems[NB:2 * NB]
    sem_w = bufs_sems[2 * NB:]
    wid = lax.axis_index("s") * NC + lax.axis_index("c")
    wbase = wid * RPW
    lane = lax.iota(jnp.int32, L)

    pltpu.sync_copy(idx_hbm.at[pl.ds(wbase, RPW)], raw_v)
    pltpu.sync_copy(bias_hbm, bias_v)
    for k in range(NCH):
        for j in range(CHUNK // L):
            pos = wbase + k * CHUNK + j * L + lane
            v = jnp.maximum(raw_v[pl.ds(k * CHUNK + j * L, L)], 0)
            gid_v[k, pl.ds(j * L, L)] = v + lax.rem(pos, F) * CARD

    gd, wd = {}, {}

    def fire(k):
        gd[k] = pltpu.async_copy(tab_hbm.at[gid_v.at[k]], bufq[k % NB], sem_g[k % NB])

    fire(0)
    fire(1)
    for k in range(NCH):
        s = k % NB
        if k + 2 < NCH:
            if k - 2 >= 0:
                wd[k - 2].wait()
            fire(k + 2)
        gd[k].wait()
        base = wbase + k * CHUNK
        buf = bufq[s]

        @pl.loop(0, CHUNK)
        def _bias_add(r):
            fr = lax.rem(base + r, F)
            for j in range(VPR):
                sl = pl.ds(j * L, L)
                buf[r, sl] = buf[r, sl] + bias_v[fr, sl]

        wd[k] = pltpu.async_copy(buf, out_hbm.at[pl.ds(base, CHUNK), :], sem_w[s])

    for k in range(max(0, NCH - 4), NCH):
        wd[k].wait()


def kernel(x_cat, tables, bias):
    idx = x_cat.astype(jnp.int32).reshape(ROWS)
    tab = tables.reshape(F * CARD, D)
    out = _tokenize(idx, tab, bias)
    return out.reshape(B, F, D)


# speedup vs baseline: 9.7776x; 1.3001x over previous
import functools

import jax
import jax.numpy as jnp
from jax import lax
from jax.experimental import pallas as pl
from jax.experimental.pallas import tpu as pltpu
from jax.experimental.pallas import tpu_sc as plsc

F = 26
CARD = 1000
D = 128
B = 4096
L = 16
NC, NS = 2, 16
NW = NC * NS
ROWS = B * F
RPW = ROWS // NW
BPW = B // NW
CHB = 4
CHUNK = CHB * F
NCH = BPW // CHB
VPR = D // L
NB = 4

_mesh = plsc.VectorSubcoreMesh(core_axis_name="c", subcore_axis_name="s")


@functools.partial(
    pl.kernel,
    out_type=jax.ShapeDtypeStruct((B, F, D), jnp.float32),
    mesh=_mesh,
    scratch_types=[
        pltpu.VMEM((RPW,), jnp.int32),
        pltpu.VMEM((F, D), jnp.float32),
    ]
    + [pltpu.VMEM((CHUNK, D), jnp.float32) for _ in range(NB)]
    + [pltpu.SemaphoreType.DMA for _ in range(2 * NB)],
)
def _tokenize(idx_hbm, tab_hbm, bias_hbm, out_hbm, gid_v, bias_v, *bufs_sems):
    bufq = bufs_sems[:NB]
    sem_g = bufs_sems[NB:2 * NB]
    sem_w = bufs_sems[2 * NB:]
    wid = lax.axis_index("s") * NC + lax.axis_index("c")
    wbase = wid * RPW
    b0w = wid * BPW
    lane = lax.iota(jnp.int32, L)

    pltpu.sync_copy(idx_hbm.at[pl.ds(wbase, RPW)], gid_v)
    pltpu.sync_copy(bias_hbm, bias_v)
    for g in range(RPW // L):
        sl = pl.ds(g * L, L)
        fvec = lax.rem(jnp.full((L,), g * L, jnp.int32) + lane, F)
        gid_v[sl] = jnp.maximum(gid_v[sl], 0) + fvec * CARD

    gd, wd = {}, {}

    def fire(k):
        gd[k] = pltpu.async_copy(
            tab_hbm.at[gid_v.at[pl.ds(k * CHUNK, CHUNK)]], bufq[k % NB], sem_g[k % NB]
        )

    fire(0)
    fire(1)
    for k in range(NCH):
        s = k % NB
        if k + 2 < NCH:
            if k - 2 >= 0:
                for d in wd[k - 2]:
                    d.wait()
            fire(k + 2)
        gd[k].wait()
        buf = bufq[s]

        @pl.loop(0, CHUNK)
        def _bias_add(r):
            fr = lax.rem(r, F)
            for j in range(VPR):
                sl = pl.ds(j * L, L)
                buf[r, sl] = buf[r, sl] + bias_v[fr, sl]

        wd[k] = [
            pltpu.async_copy(
                buf.at[pl.ds(r * F, F)], out_hbm.at[b0w + k * CHB + r], sem_w[s]
            )
            for r in range(CHB)
        ]

    for k in range(max(0, NCH - 4), NCH):
        for d in wd[k]:
            d.wait()


def kernel(x_cat, tables, bias):
    idx = x_cat.astype(jnp.int32).reshape(ROWS)
    tab = tables.reshape(F * CARD, D)
    return _tokenize(idx, tab, bias)


# speedup vs baseline: 9.7884x; 1.0011x over previous
import functools

import jax
import jax.numpy as jnp
from jax import lax
from jax.experimental import pallas as pl
from jax.experimental.pallas import tpu as pltpu
from jax.experimental.pallas import tpu_sc as plsc

F = 26
CARD = 1000
D = 128
B = 4096
L = 16
NC, NS = 2, 16
NW = NC * NS
ROWS = B * F
RPW = ROWS // NW
BPW = B // NW
CHB = 4
CHUNK = CHB * F
NCH = BPW // CHB
VPR = D // L
NB = 4

_mesh = plsc.VectorSubcoreMesh(core_axis_name="c", subcore_axis_name="s")


@functools.partial(
    pl.kernel,
    out_type=jax.ShapeDtypeStruct((B, F, D), jnp.float32),
    mesh=_mesh,
    scratch_types=[
        pltpu.VMEM((RPW,), jnp.int32),
        pltpu.VMEM((F, D), jnp.float32),
    ]
    + [pltpu.VMEM((CHUNK, D), jnp.float32) for _ in range(NB)]
    + [pltpu.SemaphoreType.DMA for _ in range(2 * NB)],
    compiler_params=pltpu.CompilerParams(use_tc_tiling_on_sc=True),
)
def _tokenize(idx_hbm, tab_hbm, bias_hbm, out_hbm, gid_v, bias_v, *bufs_sems):
    bufq = bufs_sems[:NB]
    sem_g = bufs_sems[NB:2 * NB]
    sem_w = bufs_sems[2 * NB:]
    wid = lax.axis_index("s") * NC + lax.axis_index("c")
    wbase = wid * RPW
    b0w = wid * BPW
    lane = lax.iota(jnp.int32, L)

    pltpu.sync_copy(idx_hbm.at[pl.ds(wbase, RPW)], gid_v)
    pltpu.sync_copy(bias_hbm, bias_v)
    for g in range(RPW // L):
        sl = pl.ds(g * L, L)
        fvec = lax.rem(jnp.full((L,), g * L, jnp.int32) + lane, F)
        gid_v[sl] = jnp.maximum(gid_v[sl], 0) + fvec * CARD

    gd, wd = {}, {}

    def fire(k):
        gd[k] = pltpu.async_copy(
            tab_hbm.at[gid_v.at[pl.ds(k * CHUNK, CHUNK)]], bufq[k % NB], sem_g[k % NB]
        )

    fire(0)
    fire(1)
    for k in range(NCH):
        s = k % NB
        if k + 2 < NCH:
            if k - 2 >= 0:
                for d in wd[k - 2]:
                    d.wait()
            fire(k + 2)
        gd[k].wait()
        buf = bufq[s]

        @pl.loop(0, CHUNK)
        def _bias_add(r):
            fr = lax.rem(r, F)
            for j in range(VPR):
                sl = pl.ds(j * L, L)
                buf[r, sl] = buf[r, sl] + bias_v[fr, sl]

        wd[k] = [
            pltpu.async_copy(
                buf.at[pl.ds(r * F, F)], out_hbm.at[b0w + k * CHB + r], sem_w[s]
            )
            for r in range(CHB)
        ]

    for k in range(max(0, NCH - 4), NCH):
        for d in wd[k]:
            d.wait()


def kernel(x_cat, tables, bias):
    idx = x_cat.astype(jnp.int32).reshape(ROWS)
    tab = tables.reshape(F * CARD, D)
    return _tokenize(idx, tab, bias)


# speedup vs baseline: 30.2749x; 3.0929x over previous
import functools

import jax
import jax.numpy as jnp
from jax import lax
from jax.experimental import pallas as pl
from jax.experimental.pallas import tpu as pltpu
from jax.experimental.pallas import tpu_sc as plsc

F = 26
CARD = 1000
D = 128
B = 4096
L = 16
NC, NS = 2, 16
NW = NC * NS
ROWS = B * F
RPW = ROWS // NW
CHUNK = 128
NCH = RPW // CHUNK
VPR = D // L
NB = 4

_mesh = plsc.VectorSubcoreMesh(core_axis_name="c", subcore_axis_name="s")


@functools.partial(
    pl.kernel,
    out_type=jax.ShapeDtypeStruct((ROWS, D), jnp.float32),
    mesh=_mesh,
    scratch_types=[
        pltpu.VMEM((RPW,), jnp.int32),
        pltpu.VMEM((F, D), jnp.float32),
    ]
    + [pltpu.VMEM((CHUNK, D), jnp.float32) for _ in range(NB)]
    + [pltpu.SemaphoreType.DMA for _ in range(2 * NB)],
)
def _tokenize(idx_hbm, tab_hbm, bias_hbm, out_hbm, gid_v, bias_v, *bufs_sems):
    bufq = bufs_sems[:NB]
    sem_g = bufs_sems[NB:2 * NB]
    sem_w = bufs_sems[2 * NB:]
    wid = lax.axis_index("s") * NC + lax.axis_index("c")
    wbase = wid * RPW
    lane = lax.iota(jnp.int32, L)

    pltpu.sync_copy(idx_hbm.at[pl.ds(wbase, RPW)], gid_v)
    pltpu.sync_copy(bias_hbm, bias_v)
    for g in range(RPW // L):
        sl = pl.ds(g * L, L)
        fvec = lax.shift_right_logical(wbase + g * L + lane, 12)
        gid_v[sl] = jnp.maximum(gid_v[sl], 0) + fvec * CARD

    gd, wd = {}, {}

    def fire(k):
        gd[k] = pltpu.async_copy(
            tab_hbm.at[gid_v.at[pl.ds(k * CHUNK, CHUNK)]], bufq[k % NB], sem_g[k % NB]
        )

    fire(0)
    fire(1)
    for k in range(NCH):
        s = k % NB
        if k + 2 < NCH:
            if k - 2 >= 0:
                wd[k - 2].wait()
            fire(k + 2)
        gd[k].wait()
        buf = bufq[s]
        fk = lax.shift_right_logical(wbase + k * CHUNK, 12)
        bvals = [bias_v[fk, pl.ds(j * L, L)] for j in range(VPR)]

        @pl.loop(0, CHUNK)
        def _bias_add(r):
            for j in range(VPR):
                sl = pl.ds(j * L, L)
                buf[r, sl] = buf[r, sl] + bvals[j]

        wd[k] = pltpu.async_copy(
            buf, out_hbm.at[pl.ds(wbase + k * CHUNK, CHUNK), :], sem_w[s]
        )

    for k in range(max(0, NCH - 4), NCH):
        wd[k].wait()


def kernel(x_cat, tables, bias):
    idx_fmajor = x_cat.astype(jnp.int32).T.reshape(ROWS)
    tab = tables.reshape(F * CARD, D)
    out = _tokenize(idx_fmajor, tab, bias)
    return out.reshape(F, B, D).transpose(1, 0, 2)
